# Initial kernel scaffold; baseline (speedup 1.0000x reference)
import jax
import jax.numpy as jnp
from jax.experimental import pallas as pl


def kernel(X, edge_index, token_table, gene_table, Wl, Wr, att, ln1_scale, ln1_bias, ffn_w1, ffn_b1, ffn_w2, ffn_b2, ln2_scale, ln2_bias, W1, b1, W2, b2):
    raise NotImplementedError("write your pallas kernel here")


# speedup vs baseline: 10.2341x; 10.2341x over previous
import jax
import jax.numpy as jnp
from jax import lax
from jax.experimental import pallas as pl
from jax.experimental.pallas import tpu as pltpu
from jax.experimental.pallas import tpu_sc as plsc

G = 10000
F = 128
H = 4
DH = 32
E = 160000
B = 2
BE = B * E

GB = 2000
NBLK = G // GB
EB = 2000
NSUB = 16
EPS = E // NSUB
CH = 80
NCHUNK = EPS // CH
GP = 10240
GPS = GP // NSUB


def _embed_proj_body(ohp_ref, ttp_ref, gene_ref, wl_ref, wr_ref,
                     x_ref, hl_ref, hr_ref):
    x = jnp.dot(ohp_ref[...], ttp_ref[...],
                preferred_element_type=jnp.float32, precision=lax.Precision.HIGHEST) + gene_ref[...]
    x_ref[...] = x
    hl_ref[...] = jnp.dot(x, wl_ref[...], preferred_element_type=jnp.float32, precision=lax.Precision.HIGHEST)
    hr_ref[...] = jnp.dot(x, wr_ref[...], preferred_element_type=jnp.float32, precision=lax.Precision.HIGHEST)


def _embed_proj(ohp, ttp, gene_table, Wl, Wr):
    grid = (B * NBLK,)
    full = pl.BlockSpec((F, F), lambda i: (0, 0))
    return pl.pallas_call(
        _embed_proj_body,
        grid=grid,
        in_specs=[
            pl.BlockSpec((GB, F), lambda i: (i, 0)),
            full,
            pl.BlockSpec((GB, F), lambda i: (i % NBLK, 0)),
            full,
            full,
        ],
        out_specs=[pl.BlockSpec((GB, F), lambda i: (i, 0))] * 3,
        out_shape=[jax.ShapeDtypeStruct((B * G, F), jnp.float32)] * 3,
    )(ohp, ttp, gene_table, Wl, Wr)


def _gather_body(hl_hbm, hr_hbm, srcg_hbm, dstg_hbm, hs_out, hd_out,
                 idxs_v, idxd_v, rows_s, rows_d, sem1, sem2):
    c = lax.axis_index("c")
    s = lax.axis_index("s")

    def chunk(ci, carry):
        base = c * E + s * EPS + ci * CH
        pltpu.sync_copy(srcg_hbm.at[pl.ds(base, CH)], idxs_v)
        pltpu.sync_copy(dstg_hbm.at[pl.ds(base, CH)], idxd_v)
        cp1 = pltpu.async_copy(hl_hbm.at[idxs_v], rows_s, sem1)
        cp2 = pltpu.async_copy(hr_hbm.at[idxd_v], rows_d, sem2)
        cp1.wait()
        cp2.wait()
        pltpu.sync_copy(rows_s, hs_out.at[pl.ds(base, CH)])
        pltpu.sync_copy(rows_d, hd_out.at[pl.ds(base, CH)])
        return carry

    lax.fori_loop(0, NCHUNK, chunk, 0)


def _gather_edges(hl2d, hr2d, srcg, dstg):
    mesh = plsc.VectorSubcoreMesh(core_axis_name="c", subcore_axis_name="s")
    fn = pl.kernel(
        _gather_body,
        out_type=[jax.ShapeDtypeStruct((BE, F), jnp.float32),
                  jax.ShapeDtypeStruct((BE, F), jnp.float32)],
        mesh=mesh,
        scratch_types=[
            pltpu.VMEM((CH,), jnp.int32),
            pltpu.VMEM((CH,), jnp.int32),
            pltpu.VMEM((CH, F), jnp.float32),
            pltpu.VMEM((CH, F), jnp.float32),
            pltpu.SemaphoreType.DMA,
            pltpu.SemaphoreType.DMA,
        ],
    )
    return fn(hl2d, hr2d, srcg, dstg)


def _edge_math_body(hs_ref, hd_ref, ap_ref, a16_ref, msg_ref, p16_ref):
    hs = hs_ref[...]
    sm = hs + hd_ref[...]
    lr = jnp.maximum(sm, sm * jnp.float32(0.2))
    pexp = jnp.exp(jnp.dot(lr, ap_ref[...],
                           preferred_element_type=jnp.float32, precision=lax.Precision.HIGHEST))
    msg_ref[...] = hs * pexp
    p16_ref[...] = jnp.exp(jnp.dot(lr, a16_ref[...],
                                   preferred_element_type=jnp.float32, precision=lax.Precision.HIGHEST))


def _edge_math(hs, hd, attmatP, attmat16):
    grid = (BE // EB,)
    return pl.pallas_call(
        _edge_math_body,
        grid=grid,
        in_specs=[
            pl.BlockSpec((EB, F), lambda i: (i, 0)),
            pl.BlockSpec((EB, F), lambda i: (i, 0)),
            pl.BlockSpec((F, F), lambda i: (0, 0)),
            pl.BlockSpec((F, 16), lambda i: (0, 0)),
        ],
        out_specs=[
            pl.BlockSpec((EB, F), lambda i: (i, 0)),
            pl.BlockSpec((EB, 16), lambda i: (i, 0)),
        ],
        out_shape=[
            jax.ShapeDtypeStruct((BE, F), jnp.float32),
            jax.ShapeDtypeStruct((BE, 16), jnp.float32),
        ],
    )(hs, hd, attmatP, attmat16)


def _layer_norm(x, scale, bias):
    mu = jnp.mean(x, axis=-1, keepdims=True)
    var = jnp.mean(jnp.square(x - mu), axis=-1, keepdims=True)
    return (x - mu) / jnp.sqrt(var + 1e-5) * scale + bias


def _tail_body(x_ref, num_ref, den_ref, exp_ref, ln1s_ref, ln1b_ref,
               w1f_ref, b1f_ref, w2f_ref, b2f_ref, ln2s_ref, ln2b_ref,
               w1r_ref, w2a_ref, w2b_ref,
               emb_ref, o1_ref, o2_ref):
    i = pl.program_id(0)
    b = i // NBLK
    denx = jnp.dot(den_ref[...], exp_ref[...],
                   preferred_element_type=jnp.float32, precision=lax.Precision.HIGHEST)
    att_out = num_ref[...] / (denx + jnp.float32(1e-30))
    x = x_ref[...] + att_out
    x = _layer_norm(x, ln1s_ref[...], ln1b_ref[...])
    ff = jnp.maximum(
        jnp.dot(x, w1f_ref[...], preferred_element_type=jnp.float32, precision=lax.Precision.HIGHEST)
        + b1f_ref[...], 0.0)
    ff = jnp.dot(ff, w2f_ref[...], preferred_element_type=jnp.float32, precision=lax.Precision.HIGHEST) \
        + b2f_ref[...]
    x = _layer_norm(x + ff, ln2s_ref[...], ln2b_ref[...])
    emb_ref[...] = x

    p1 = jnp.sum(x * w1r_ref[...])
    p2a = jnp.sum(x * w2a_ref[...])
    p2b = jnp.sum(x * w2b_ref[...])

    @pl.when(i % NBLK == 0)
    def _():
        rm = lax.broadcasted_iota(jnp.int32, (B, 1), 0) == b
        o1_ref[...] = jnp.where(rm, 0.0, o1_ref[...])
        rm2 = lax.broadcasted_iota(jnp.int32, (B, 2), 0) == b
        o2_ref[...] = jnp.where(rm2, 0.0, o2_ref[...])

    rm = lax.broadcasted_iota(jnp.int32, (B, 1), 0) == b
    o1_ref[...] = o1_ref[...] + jnp.where(rm, p1, 0.0)
    rowb = lax.broadcasted_iota(jnp.int32, (B, 2), 0) == b
    colj = lax.broadcasted_iota(jnp.int32, (B, 2), 1)
    o2_ref[...] = o2_ref[...] + jnp.where(
        rowb, jnp.where(colj == 0, p2a, p2b), 0.0)


def _tail(x2d, num2d, den2d, expand, ln1s, ln1b, w1f, b1f, w2f, b2f,
          ln2s, ln2b, w1r, w2a, w2b):
    grid = (B * NBLK,)
    full = pl.BlockSpec((F, F), lambda i: (0, 0))
    row = pl.BlockSpec((1, F), lambda i: (0, 0))
    gblk = pl.BlockSpec((GB, F), lambda i: (i % NBLK, 0))
    return pl.pallas_call(
        _tail_body,
        grid=grid,
        in_specs=[
            pl.BlockSpec((GB, F), lambda i: (i, 0)),
            pl.BlockSpec((GB, F), lambda i: (i, 0)),
            pl.BlockSpec((GB, 16), lambda i: (i, 0)),
            pl.BlockSpec((16, F), lambda i: (0, 0)),
            row, row,
            full, row, full, row,
            row, row,
            gblk, gblk, gblk,
        ],
        out_specs=[
            pl.BlockSpec((GB, F), lambda i: (i, 0)),
            pl.BlockSpec((B, 1), lambda i: (0, 0)),
            pl.BlockSpec((B, 2), lambda i: (0, 0)),
        ],
        out_shape=[
            jax.ShapeDtypeStruct((B * G, F), jnp.float32),
            jax.ShapeDtypeStruct((B, 1), jnp.float32),
            jax.ShapeDtypeStruct((B, 2), jnp.float32),
        ],
    )(x2d, num2d, den2d, expand, ln1s, ln1b, w1f, b1f, w2f, b2f,
      ln2s, ln2b, w1r, w2a, w2b)


def kernel(X, edge_index, token_table, gene_table, Wl, Wr, att,
           ln1_scale, ln1_bias, ffn_w1, ffn_b1, ffn_w2, ffn_b2,
           ln2_scale, ln2_bias, W1, b1, W2, b2):
    f32 = jnp.float32
    Xf = X.reshape(-1)
    oh = (Xf[:, None] == jnp.arange(4, dtype=Xf.dtype)).astype(f32)
    ohp = jnp.pad(oh, ((0, 0), (0, F - 4)))
    ttp = jnp.pad(token_table.astype(f32), ((0, F - 4), (0, 0)))

    x2d, hl2d, hr2d = _embed_proj(ohp, ttp, gene_table.astype(f32),
                                  Wl.astype(f32), Wr.astype(f32))

    src = edge_index[0]
    dst = edge_index[1]
    srcg = jnp.concatenate([src, src + G])
    dstg = jnp.concatenate([dst, dst + G])
    hs, hd = _gather_edges(hl2d, hr2d, srcg, dstg)

    attf = att.reshape(-1).astype(f32)
    fi = lax.broadcasted_iota(jnp.int32, (F, F), 0)
    fj = lax.broadcasted_iota(jnp.int32, (F, F), 1)
    attmatP = jnp.where(fi // DH == fj // DH, attf[:, None], 0.0)
    hi = lax.broadcasted_iota(jnp.int32, (F, 16), 0)
    hj = lax.broadcasted_iota(jnp.int32, (F, 16), 1)
    attmat16 = jnp.where(hi // DH == hj, attf[:, None], 0.0)

    msg, p16 = _edge_math(hs, hd, attmatP, attmat16)

    num2d = jax.ops.segment_sum(msg, dstg, num_segments=B * G)
    den2d = jax.ops.segment_sum(p16, dstg, num_segments=B * G)

    expand = (lax.broadcasted_iota(jnp.int32, (16, F), 1) // DH
              == lax.broadcasted_iota(jnp.int32, (16, F), 0)).astype(f32)
    w1r = W1.reshape(G, F)
    w2r = W2.reshape(G, F, 2)
    w2a = w2r[:, :, 0]
    w2b = w2r[:, :, 1]
    w1f = jnp.pad(ffn_w1.astype(f32), ((0, 0), (0, F - ffn_w1.shape[1])))
    b1f = jnp.pad(ffn_b1.astype(f32), (0, F - ffn_b1.shape[0]))[None, :]
    w2f = jnp.pad(ffn_w2.astype(f32), ((0, F - ffn_w2.shape[0]), (0, 0)))
    b2f = ffn_b2.astype(f32)[None, :]

    emb2d, o1, o2 = _tail(
        x2d, num2d, den2d, expand,
        ln1_scale.astype(f32)[None, :], ln1_bias.astype(f32)[None, :],
        w1f, b1f, w2f, b2f,
        ln2_scale.astype(f32)[None, :], ln2_bias.astype(f32)[None, :],
        w1r, w2a, w2b)

    out1 = o1 + b1[None, :]
    out2 = o2 + b2[None, :]
    gene_emb = emb2d.reshape(B, G, F)
    return out1, out2, gene_emb
